# Initial kernel scaffold; baseline (speedup 1.0000x reference)
import jax
import jax.numpy as jnp
from jax.experimental import pallas as pl


def kernel(h, segment_ids):
    raise NotImplementedError("write your pallas kernel here")


# speedup vs baseline: 6.0329x; 6.0329x over previous
import functools

import jax
import jax.numpy as jnp
from jax import lax
from jax.experimental import pallas as pl
from jax.experimental.pallas import tpu as pltpu
from jax.experimental.pallas import tpu_sc as plsc

N = 100000
D = 128
S = 128
NW = 32
CH = 3136
T = 224
NT = CH // T
NV = D // 16
G = 16
NG = T // G


def _sc_partials(h_flat, ids):
    mesh = plsc.VectorSubcoreMesh(core_axis_name="c", subcore_axis_name="s")

    @functools.partial(
        pl.kernel,
        mesh=mesh,
        out_type=jax.ShapeDtypeStruct((NW * S * D,), jnp.float32),
        scratch_types=[
            pltpu.VMEM((CH,), jnp.int32),
            pltpu.VMEM((T * D,), jnp.float32),
            pltpu.VMEM((T * D,), jnp.float32),
            pltpu.VMEM((S * D,), jnp.float32),
            pltpu.SemaphoreType.DMA,
            pltpu.SemaphoreType.DMA,
        ],
    )
    def k(h_hbm, ids_hbm, out_hbm, ids_v, buf0, buf1, acc_v, sem0, sem1):
        wid = lax.axis_index("s") * 2 + lax.axis_index("c")
        base = ((wid * (N - CH)) // (NW - 1)) // 8 * 8
        base = pl.multiple_of(base, 8)
        bufs = (buf0, buf1)
        sems = (sem0, sem1)

        pltpu.sync_copy(ids_hbm.at[pl.ds(base, CH)], ids_v)

        neg = jnp.full((16,), -jnp.inf, dtype=jnp.float32)

        def init_blk(i, c):
            acc_v[pl.ds(i * 16, 16)] = neg
            return c

        lax.fori_loop(0, S * D // 16, init_blk, 0)

        def start_copy(t, b):
            pltpu.async_copy(
                h_hbm.at[pl.ds((base + t * T) * D, T * D)], bufs[b], sems[b]
            )

        def wait_copy(t, b):
            pltpu.make_async_copy(
                h_hbm.at[pl.ds((base + t * T) * D, T * D)], bufs[b], sems[b]
            ).wait()

        def process(t, b, carry):
            @pl.when(t + 1 < NT)
            def _():
                start_copy(t + 1, 1 - b)

            wait_copy(t, b)
            buf = bufs[b]

            def group(j, c):
                idv = ids_v[pl.ds(t * T + j * G, G)]
                for r in range(G):
                    prev = c[0]
                    accs = c[1:]
                    sid = idv[r]
                    flush = sid != prev

                    @pl.when(flush)
                    def _(prev=prev, accs=accs):
                        for v in range(NV):
                            acc_v[pl.ds(prev * D + v * 16, 16)] = accs[v]

                    new = []
                    for v in range(NV):
                        rv = buf[pl.ds((j * G + r) * D + v * 16, 16)]
                        m = jnp.maximum(accs[v], rv)
                        new.append(jnp.where(flush, rv, m))
                    c = (sid, *new)
                return c

            return lax.fori_loop(0, NG, group, carry)

        start_copy(0, 0)
        carry = (ids_v[pl.ds(0, 16)][0],) + (neg,) * NV

        def pair(t, c):
            g = 2 * t
            c = process(g, 0, c)
            c = process(g + 1, 1, c)
            return c

        carry = lax.fori_loop(0, NT // 2, pair, carry)

        prev = carry[0]
        for v in range(NV):
            acc_v[pl.ds(prev * D + v * 16, 16)] = carry[1 + v]

        pltpu.sync_copy(acc_v, out_hbm.at[pl.ds(wid * S * D, S * D)])

    return k(h_flat, ids)


def _merge(partials):
    def body(p_ref, o_ref):
        o_ref[...] = jnp.max(p_ref[...], axis=0)

    return pl.pallas_call(
        body,
        out_shape=jax.ShapeDtypeStruct((S, D), jnp.float32),
    )(partials)


def kernel(h, segment_ids):
    partials = _sc_partials(h.reshape(N * D), segment_ids)
    return _merge(partials.reshape(NW, S, D))
